# baseline (speedup 1.0000x reference)
import functools
import math

import jax
import jax.numpy as jnp
from jax.experimental import pallas as pl

_CLIP_RATIO = 0.016
_MAX_RATIO = abs(math.log(_CLIP_RATIO))


def _body(score_ref, logits_ref, regress_ref, anchors_ref,
          logits_out_ref, boxes_out_ref):
    s = score_ref[...]
    logits_out_ref[...] = logits_ref[...] * s

    a = anchors_ref[...]
    r = regress_ref[...]
    acx, acy = a[:, 0:1], a[:, 1:2]
    aw, ah = a[:, 2:3], a[:, 3:4]
    cx = acx + r[:, 0:1] * aw
    cy = acy + r[:, 1:2] * ah
    dw = jnp.clip(r[:, 2:3], -_MAX_RATIO, _MAX_RATIO)
    dh = jnp.clip(r[:, 3:4], -_MAX_RATIO, _MAX_RATIO)
    w = aw * jnp.exp(dw)
    h = ah * jnp.exp(dh)
    x1 = cx - 0.5 * w
    y1 = cy - 0.5 * h
    x2 = cx + 0.5 * w
    y2 = cy + 0.5 * h
    boxes = jnp.concatenate([x1, y1, x2, y2], axis=-1)
    boxes_out_ref[...] = jnp.clip(boxes, 0.0, 1.0)


@functools.partial(jax.jit, static_argnames=("interpret",))
def kernel(score, logits, regress, anchors, interpret=False):
    B, N, C = logits.shape
    BN = 2000
    rows = B * N
    n_blocks = N // BN
    grid = (rows // BN,)

    score2 = score.reshape(rows, 1)
    logits2 = logits.reshape(rows, C)
    regress2 = regress.reshape(rows, 4)

    logits_out, boxes_out = pl.pallas_call(
        _body,
        grid=grid,
        in_specs=[
            pl.BlockSpec((BN, 1), lambda i: (i, 0)),
            pl.BlockSpec((BN, C), lambda i: (i, 0)),
            pl.BlockSpec((BN, 4), lambda i: (i, 0)),
            pl.BlockSpec((BN, 4), lambda i: (i % n_blocks, 0)),
        ],
        out_specs=[
            pl.BlockSpec((BN, C), lambda i: (i, 0)),
            pl.BlockSpec((BN, 4), lambda i: (i, 0)),
        ],
        out_shape=[
            jax.ShapeDtypeStruct((rows, C), jnp.float32),
            jax.ShapeDtypeStruct((rows, 4), jnp.float32),
        ],
        interpret=interpret,
    )(score2, logits2, regress2, anchors)

    return logits_out.reshape(B, N, C), boxes_out.reshape(B, N, 4)


# speedup vs baseline: 1.1449x; 1.1449x over previous
import functools
import math

import jax
import jax.numpy as jnp
from jax.experimental import pallas as pl

_CLIP_RATIO = 0.016
_MAX_RATIO = abs(math.log(_CLIP_RATIO))


def _body(score_ref, logits_ref, regress_ref, anchors_ref,
          logits_out_ref, boxes_out_ref):
    s = score_ref[...]
    logits_out_ref[...] = logits_ref[...] * s

    a = anchors_ref[...]
    r = regress_ref[0]
    acx, acy = a[:, 0:1], a[:, 1:2]
    aw, ah = a[:, 2:3], a[:, 3:4]
    cx = acx + r[:, 0:1] * aw
    cy = acy + r[:, 1:2] * ah
    dw = jnp.clip(r[:, 2:3], -_MAX_RATIO, _MAX_RATIO)
    dh = jnp.clip(r[:, 3:4], -_MAX_RATIO, _MAX_RATIO)
    w = aw * jnp.exp(dw)
    h = ah * jnp.exp(dh)
    x1 = cx - 0.5 * w
    y1 = cy - 0.5 * h
    x2 = cx + 0.5 * w
    y2 = cy + 0.5 * h
    boxes = jnp.concatenate([x1, y1, x2, y2], axis=-1)
    boxes_out_ref[0] = jnp.clip(boxes, 0.0, 1.0)


@functools.partial(jax.jit, static_argnames=("interpret",))
def kernel(score, logits, regress, anchors, interpret=False):
    B, N, C = logits.shape
    BN = 2000

    logits_out, boxes_out = pl.pallas_call(
        _body,
        grid=(N // BN, B),
        in_specs=[
            pl.BlockSpec((1, BN, 1), lambda i, b: (b, i, 0)),
            pl.BlockSpec((1, BN, C), lambda i, b: (b, i, 0)),
            pl.BlockSpec((1, BN, 4), lambda i, b: (b, i, 0)),
            pl.BlockSpec((BN, 4), lambda i, b: (i, 0)),
        ],
        out_specs=[
            pl.BlockSpec((1, BN, C), lambda i, b: (b, i, 0)),
            pl.BlockSpec((1, BN, 4), lambda i, b: (b, i, 0)),
        ],
        out_shape=[
            jax.ShapeDtypeStruct((B, N, C), jnp.float32),
            jax.ShapeDtypeStruct((B, N, 4), jnp.float32),
        ],
        interpret=interpret,
    )(score, logits, regress, anchors)

    return logits_out, boxes_out


# speedup vs baseline: 9.1354x; 7.9795x over previous
import functools
import math

import jax
import jax.numpy as jnp
from jax.experimental import pallas as pl

_CLIP_RATIO = 0.016
_MAX_RATIO = abs(math.log(_CLIP_RATIO))


def _body(score_ref, logits_ref, regress_ref, anchors_ref,
          logits_out_ref, boxes_out_ref):
    b = pl.program_id(0)
    c = pl.program_id(1)

    s = score_ref[pl.ds(b, 1), :]
    logits_out_ref[...] = logits_ref[...] * s[None]

    @pl.when(c == 0)
    def _boxes():
        a = anchors_ref[...]
        r = regress_ref[0]
        cx = a[0:1] + r[0:1] * a[2:3]
        cy = a[1:2] + r[1:2] * a[3:4]
        w = a[2:3] * jnp.exp(jnp.clip(r[2:3], -_MAX_RATIO, _MAX_RATIO))
        h = a[3:4] * jnp.exp(jnp.clip(r[3:4], -_MAX_RATIO, _MAX_RATIO))
        x1 = cx - 0.5 * w
        y1 = cy - 0.5 * h
        x2 = cx + 0.5 * w
        y2 = cy + 0.5 * h
        boxes = jnp.concatenate([x1, y1, x2, y2], axis=0)
        boxes_out_ref[0] = jnp.clip(boxes, 0.0, 1.0)


@functools.partial(jax.jit, static_argnames=("interpret",))
def kernel(score, logits, regress, anchors, interpret=False):
    B, N, C = logits.shape
    CB = 16

    logits_t = logits.transpose(0, 2, 1)
    score2 = score[:, :, 0]
    regress_t = regress.transpose(0, 2, 1)
    anchors_t = anchors.transpose(1, 0)

    logits_out_t, boxes_out_t = pl.pallas_call(
        _body,
        grid=(B, C // CB),
        in_specs=[
            pl.BlockSpec((B, N), lambda b, c: (0, 0)),
            pl.BlockSpec((1, CB, N), lambda b, c: (b, c, 0)),
            pl.BlockSpec((1, 4, N), lambda b, c: (b, 0, 0)),
            pl.BlockSpec((4, N), lambda b, c: (0, 0)),
        ],
        out_specs=[
            pl.BlockSpec((1, CB, N), lambda b, c: (b, c, 0)),
            pl.BlockSpec((1, 4, N), lambda b, c: (b, 0, 0)),
        ],
        out_shape=[
            jax.ShapeDtypeStruct((B, C, N), jnp.float32),
            jax.ShapeDtypeStruct((B, 4, N), jnp.float32),
        ],
        interpret=interpret,
    )(score2, logits_t, regress_t, anchors_t)

    return logits_out_t.transpose(0, 2, 1), boxes_out_t.transpose(0, 2, 1)


# speedup vs baseline: 11.6601x; 1.2764x over previous
import functools
import math

import jax
import jax.numpy as jnp
from jax.experimental import pallas as pl

_CLIP_RATIO = 0.016
_MAX_RATIO = abs(math.log(_CLIP_RATIO))


def _body(score_ref, logits_ref, regress_ref, anchors_ref,
          logits_out_ref, boxes_out_ref):
    b = pl.program_id(0)
    c = pl.program_id(1)

    s = score_ref[pl.ds(b, 1), :]
    logits_out_ref[...] = logits_ref[...] * s[None]

    @pl.when(c == 0)
    def _boxes():
        a = anchors_ref[...]
        r = regress_ref[0]
        cx = a[0:1] + r[0:1] * a[2:3]
        cy = a[1:2] + r[1:2] * a[3:4]
        w = a[2:3] * jnp.exp(jnp.clip(r[2:3], -_MAX_RATIO, _MAX_RATIO))
        h = a[3:4] * jnp.exp(jnp.clip(r[3:4], -_MAX_RATIO, _MAX_RATIO))
        x1 = cx - 0.5 * w
        y1 = cy - 0.5 * h
        x2 = cx + 0.5 * w
        y2 = cy + 0.5 * h
        boxes = jnp.concatenate([x1, y1, x2, y2], axis=0)
        boxes_out_ref[0] = jnp.clip(boxes, 0.0, 1.0)


@functools.partial(jax.jit, static_argnames=("interpret",))
def kernel(score, logits, regress, anchors, interpret=False):
    B, N, C = logits.shape
    CB = 40

    logits_t = logits.transpose(0, 2, 1)
    score2 = score[:, :, 0]
    regress_t = regress.transpose(0, 2, 1)
    anchors_t = anchors.transpose(1, 0)

    logits_out_t, boxes_out_t = pl.pallas_call(
        _body,
        grid=(B, C // CB),
        in_specs=[
            pl.BlockSpec((B, N), lambda b, c: (0, 0)),
            pl.BlockSpec((1, CB, N), lambda b, c: (b, c, 0)),
            pl.BlockSpec((1, 4, N), lambda b, c: (b, 0, 0)),
            pl.BlockSpec((4, N), lambda b, c: (0, 0)),
        ],
        out_specs=[
            pl.BlockSpec((1, CB, N), lambda b, c: (b, c, 0)),
            pl.BlockSpec((1, 4, N), lambda b, c: (b, 0, 0)),
        ],
        out_shape=[
            jax.ShapeDtypeStruct((B, C, N), jnp.float32),
            jax.ShapeDtypeStruct((B, 4, N), jnp.float32),
        ],
        interpret=interpret,
    )(score2, logits_t, regress_t, anchors_t)

    return logits_out_t.transpose(0, 2, 1), boxes_out_t.transpose(0, 2, 1)


# speedup vs baseline: 12.2762x; 1.0528x over previous
import functools
import math

import jax
import jax.numpy as jnp
from jax.experimental import pallas as pl

_CLIP_RATIO = 0.016
_MAX_RATIO = abs(math.log(_CLIP_RATIO))


def _body(score_ref, logits_ref, regress_ref, anchors_ref,
          logits_out_ref, boxes_out_ref):
    b = pl.program_id(0)
    c = pl.program_id(1)

    s = score_ref[pl.ds(b, 1), :]
    logits_out_ref[...] = logits_ref[...] * s[None]

    @pl.when(c == 0)
    def _boxes():
        a = anchors_ref[...]
        r = regress_ref[0]
        cx = a[0:1] + r[0:1] * a[2:3]
        cy = a[1:2] + r[1:2] * a[3:4]
        w = a[2:3] * jnp.exp(jnp.clip(r[2:3], -_MAX_RATIO, _MAX_RATIO))
        h = a[3:4] * jnp.exp(jnp.clip(r[3:4], -_MAX_RATIO, _MAX_RATIO))
        x1 = cx - 0.5 * w
        y1 = cy - 0.5 * h
        x2 = cx + 0.5 * w
        y2 = cy + 0.5 * h
        boxes = jnp.concatenate([x1, y1, x2, y2], axis=0)
        boxes_out_ref[0] = jnp.clip(boxes, 0.0, 1.0)


@functools.partial(jax.jit, static_argnames=("interpret",))
def kernel(score, logits, regress, anchors, interpret=False):
    B, N, C = logits.shape
    CB = 80

    logits_t = logits.transpose(0, 2, 1)
    score2 = score[:, :, 0]
    regress_t = regress.transpose(0, 2, 1)
    anchors_t = anchors.transpose(1, 0)

    logits_out_t, boxes_out_t = pl.pallas_call(
        _body,
        grid=(B, C // CB),
        in_specs=[
            pl.BlockSpec((B, N), lambda b, c: (0, 0)),
            pl.BlockSpec((1, CB, N), lambda b, c: (b, c, 0)),
            pl.BlockSpec((1, 4, N), lambda b, c: (b, 0, 0)),
            pl.BlockSpec((4, N), lambda b, c: (0, 0)),
        ],
        out_specs=[
            pl.BlockSpec((1, CB, N), lambda b, c: (b, c, 0)),
            pl.BlockSpec((1, 4, N), lambda b, c: (b, 0, 0)),
        ],
        out_shape=[
            jax.ShapeDtypeStruct((B, C, N), jnp.float32),
            jax.ShapeDtypeStruct((B, 4, N), jnp.float32),
        ],
        interpret=interpret,
    )(score2, logits_t, regress_t, anchors_t)

    return logits_out_t.transpose(0, 2, 1), boxes_out_t.transpose(0, 2, 1)


# speedup vs baseline: 12.8134x; 1.0438x over previous
import functools
import math

import jax
import jax.numpy as jnp
from jax.experimental import pallas as pl

_CLIP_RATIO = 0.016
_MAX_RATIO = abs(math.log(_CLIP_RATIO))


def _body(score_ref, logits_ref, regress_ref, anchors_ref,
          logits_out_ref, boxes_out_ref):
    c = pl.program_id(1)

    s = score_ref[0]
    logits_out_ref[...] = logits_ref[...] * s[None]

    @pl.when(c == 0)
    def _boxes():
        a = anchors_ref[...]
        r = regress_ref[0]
        cx = a[0:1] + r[0:1] * a[2:3]
        cy = a[1:2] + r[1:2] * a[3:4]
        w = a[2:3] * jnp.exp(jnp.clip(r[2:3], -_MAX_RATIO, _MAX_RATIO))
        h = a[3:4] * jnp.exp(jnp.clip(r[3:4], -_MAX_RATIO, _MAX_RATIO))
        x1 = cx - 0.5 * w
        y1 = cy - 0.5 * h
        x2 = cx + 0.5 * w
        y2 = cy + 0.5 * h
        boxes = jnp.concatenate([x1, y1, x2, y2], axis=0)
        boxes_out_ref[0] = jnp.clip(boxes, 0.0, 1.0)


@functools.partial(jax.jit, static_argnames=("interpret",))
def kernel(score, logits, regress, anchors, interpret=False):
    B, N, C = logits.shape
    CB = 80

    logits_t = logits.transpose(0, 2, 1)
    score_t = score.transpose(0, 2, 1)
    regress_t = regress.transpose(0, 2, 1)
    anchors_t = anchors.transpose(1, 0)

    logits_out_t, boxes_out_t = pl.pallas_call(
        _body,
        grid=(B, C // CB),
        in_specs=[
            pl.BlockSpec((1, 1, N), lambda b, c: (b, 0, 0)),
            pl.BlockSpec((1, CB, N), lambda b, c: (b, c, 0)),
            pl.BlockSpec((1, 4, N), lambda b, c: (b, 0, 0)),
            pl.BlockSpec((4, N), lambda b, c: (0, 0)),
        ],
        out_specs=[
            pl.BlockSpec((1, CB, N), lambda b, c: (b, c, 0)),
            pl.BlockSpec((1, 4, N), lambda b, c: (b, 0, 0)),
        ],
        out_shape=[
            jax.ShapeDtypeStruct((B, C, N), jnp.float32),
            jax.ShapeDtypeStruct((B, 4, N), jnp.float32),
        ],
        interpret=interpret,
    )(score_t, logits_t, regress_t, anchors_t)

    return logits_out_t.transpose(0, 2, 1), boxes_out_t.transpose(0, 2, 1)


# speedup vs baseline: 13.0746x; 1.0204x over previous
import functools
import math

import jax
import jax.numpy as jnp
from jax.experimental import pallas as pl

_CLIP_RATIO = 0.016
_MAX_RATIO = abs(math.log(_CLIP_RATIO))


def _body(score_ref, logits_ref, regress_ref, anchors_ref,
          logits_out_ref, boxes_out_ref):
    nb = logits_ref.shape[0]
    for i in range(nb):
        s = score_ref[i]
        logits_out_ref[i] = logits_ref[i] * s

    a = anchors_ref[...]
    for i in range(nb):
        r = regress_ref[i]
        cx = a[0:1] + r[0:1] * a[2:3]
        cy = a[1:2] + r[1:2] * a[3:4]
        w = a[2:3] * jnp.exp(jnp.clip(r[2:3], -_MAX_RATIO, _MAX_RATIO))
        h = a[3:4] * jnp.exp(jnp.clip(r[3:4], -_MAX_RATIO, _MAX_RATIO))
        x1 = cx - 0.5 * w
        y1 = cy - 0.5 * h
        x2 = cx + 0.5 * w
        y2 = cy + 0.5 * h
        boxes = jnp.concatenate([x1, y1, x2, y2], axis=0)
        boxes_out_ref[i] = jnp.clip(boxes, 0.0, 1.0)


@functools.partial(jax.jit, static_argnames=("interpret",))
def kernel(score, logits, regress, anchors, interpret=False):
    B, N, C = logits.shape
    BB = 2

    logits_t = logits.transpose(0, 2, 1)
    score_t = score.transpose(0, 2, 1)
    regress_t = regress.transpose(0, 2, 1)
    anchors_t = anchors.transpose(1, 0)

    logits_out_t, boxes_out_t = pl.pallas_call(
        _body,
        grid=(B // BB,),
        in_specs=[
            pl.BlockSpec((BB, 1, N), lambda b: (b, 0, 0)),
            pl.BlockSpec((BB, C, N), lambda b: (b, 0, 0)),
            pl.BlockSpec((BB, 4, N), lambda b: (b, 0, 0)),
            pl.BlockSpec((4, N), lambda b: (0, 0)),
        ],
        out_specs=[
            pl.BlockSpec((BB, C, N), lambda b: (b, 0, 0)),
            pl.BlockSpec((BB, 4, N), lambda b: (b, 0, 0)),
        ],
        out_shape=[
            jax.ShapeDtypeStruct((B, C, N), jnp.float32),
            jax.ShapeDtypeStruct((B, 4, N), jnp.float32),
        ],
        interpret=interpret,
    )(score_t, logits_t, regress_t, anchors_t)

    return logits_out_t.transpose(0, 2, 1), boxes_out_t.transpose(0, 2, 1)
